# baseline (speedup 1.0000x reference)
import functools

import jax
import jax.numpy as jnp
from jax.experimental import pallas as pl

_B = 4096
_KB = 1000
_KB_PAD = 1024
_TOPK = 5
_TEMP_INV = 1.0 / 0.07
_BLK = 512
_NBLK = _B // _BLK
_NEG = -1e30


def _prep_kern(keys_ref, sem_ref, knt_ref, semn_ref):
    k = keys_ref[...]
    kn = k / jnp.maximum(
        jnp.sqrt(jnp.sum(k * k, axis=-1, keepdims=True)), 1e-12)
    knt_ref[...] = kn.T
    s = sem_ref[...]
    semn_ref[...] = s / jnp.maximum(
        jnp.sqrt(jnp.sum(s * s, axis=-1, keepdims=True)), 1e-12)


def _main_kern(x_ref, c_ref, wt_ref, b_ref, knt_ref, vs_ref,
               e_ref, idx_ref, cost_ref):
    i = pl.program_id(0)
    x = x_ref[...]
    q = jnp.dot(x, wt_ref[...], preferred_element_type=jnp.float32) + b_ref[...]
    q = jnp.maximum(q, 0.0)
    qn = q / jnp.maximum(
        jnp.sqrt(jnp.sum(q * q, axis=-1, keepdims=True)), 1e-12)

    s = jnp.dot(qn, knt_ref[...], preferred_element_type=jnp.float32) * _TEMP_INV
    col = jax.lax.broadcasted_iota(jnp.int32, s.shape, 1)
    s = jnp.where(col < _KB, s, _NEG)

    walpha = jnp.zeros_like(s)
    denom = jnp.zeros((s.shape[0], 1), jnp.float32)
    idx_cols = []
    m0 = None
    s_cur = s
    for j in range(_TOPK):
        m = jnp.max(s_cur, axis=-1, keepdims=True)
        idx = jnp.min(jnp.where(s_cur == m, col, jnp.int32(1 << 30)),
                      axis=-1, keepdims=True)
        onehot = (col == idx)
        if j == 0:
            m0 = m
        w = jnp.exp(m - m0)
        walpha = walpha + jnp.where(onehot, w, 0.0)
        denom = denom + w
        idx_cols.append(idx)
        s_cur = jnp.where(onehot, _NEG, s_cur)
    walpha = walpha / denom

    eg = jnp.dot(walpha, vs_ref[...], preferred_element_type=jnp.float32)
    dv = e_ref.shape[1]
    e_ref[...] = eg[:, :dv]
    g = eg[:, dv:]
    idx_ref[...] = jnp.concatenate(
        idx_cols + [jnp.zeros((s.shape[0], 8 - _TOPK), jnp.int32)], axis=1)

    cs = c_ref[...]
    csn = cs / jnp.maximum(
        jnp.sqrt(jnp.sum(cs * cs, axis=-1, keepdims=True)), 1e-12)
    part = jnp.sum(1.0 - jnp.sum(csn * g, axis=-1)) * (1.0 / _B)

    @pl.when(i == 0)
    def _():
        cost_ref[...] = jnp.zeros_like(cost_ref)

    cost_ref[...] += part


@jax.jit
def kernel(u_X, c_S, W, b, keys, values, semantic_embeddings):
    x = jnp.concatenate([u_X, c_S], axis=-1)
    wt = W.T
    b2 = b.reshape(1, -1)
    pad = _KB_PAD - _KB
    keys_p = jnp.pad(keys, ((0, pad), (0, 0)))
    sem_p = jnp.pad(semantic_embeddings, ((0, pad), (0, 0)))

    d = x.shape[1]
    dk = keys.shape[1]
    dv = values.shape[1]
    dsem = semantic_embeddings.shape[1]

    knt, semn = pl.pallas_call(
        _prep_kern,
        out_shape=[
            jax.ShapeDtypeStruct((dk, _KB_PAD), jnp.float32),
            jax.ShapeDtypeStruct((_KB_PAD, dsem), jnp.float32),
        ],
    )(keys_p, sem_p)

    values_p = jnp.pad(values, ((0, pad), (0, 0)))
    vs = jnp.concatenate([values_p, semn], axis=1)

    e_out, idx_out, cost_out = pl.pallas_call(
        _main_kern,
        grid=(_NBLK,),
        in_specs=[
            pl.BlockSpec((_BLK, d), lambda i: (i, 0)),
            pl.BlockSpec((_BLK, c_S.shape[1]), lambda i: (i, 0)),
            pl.BlockSpec((d, W.shape[0]), lambda i: (0, 0)),
            pl.BlockSpec((1, W.shape[0]), lambda i: (0, 0)),
            pl.BlockSpec((dk, _KB_PAD), lambda i: (0, 0)),
            pl.BlockSpec((_KB_PAD, dv + dsem), lambda i: (0, 0)),
        ],
        out_specs=[
            pl.BlockSpec((_BLK, dv), lambda i: (i, 0)),
            pl.BlockSpec((_BLK, 8), lambda i: (i, 0)),
            pl.BlockSpec((1, 1), lambda i: (0, 0)),
        ],
        out_shape=[
            jax.ShapeDtypeStruct((_B, dv), jnp.float32),
            jax.ShapeDtypeStruct((_B, 8), jnp.int32),
            jax.ShapeDtypeStruct((1, 1), jnp.float32),
        ],
    )(x, c_S, wt, b2, knt, vs)

    return (e_out, idx_out[:, :_TOPK], cost_out[0, 0])


# speedup vs baseline: 1.5174x; 1.5174x over previous
import jax
import jax.numpy as jnp
from jax.experimental import pallas as pl

_B = 4096
_KB = 1000
_KB_PAD = 1024
_TOPK = 5
_TEMP_INV = 1.0 / 0.07
_BLK = 512
_NBLK = _B // _BLK
_NEG = -1e30


def _prep_kern(keys_ref, sem_ref, val_ref, kn_ref, vs_ref):
    k = keys_ref[...]
    kn = k / jnp.maximum(
        jnp.sqrt(jnp.sum(k * k, axis=-1, keepdims=True)), 1e-12)
    kn_ref[...] = jnp.zeros_like(kn_ref)
    kn_ref[:_KB, :] = kn
    s = sem_ref[...]
    semn = s / jnp.maximum(
        jnp.sqrt(jnp.sum(s * s, axis=-1, keepdims=True)), 1e-12)
    vs_ref[...] = jnp.zeros_like(vs_ref)
    vs_ref[:_KB, :512] = val_ref[...]
    vs_ref[:_KB, 512:] = semn


def _main_kern(u_ref, c_ref, w_ref, b_ref, kn_ref, vs_ref,
               e_ref, idx_ref, cost_ref):
    i = pl.program_id(0)
    w = w_ref[...]
    c = c_ref[...]
    nt = (((1,), (1,)), ((), ()))
    q = (jax.lax.dot_general(u_ref[...], w[:, :512], nt,
                             preferred_element_type=jnp.float32)
         + jax.lax.dot_general(c, w[:, 512:], nt,
                               preferred_element_type=jnp.float32)
         + b_ref[...])
    q = jnp.maximum(q, 0.0)
    qn = q / jnp.maximum(
        jnp.sqrt(jnp.sum(q * q, axis=-1, keepdims=True)), 1e-12)

    s = jax.lax.dot_general(qn, kn_ref[...], nt,
                            preferred_element_type=jnp.float32) * _TEMP_INV
    colf = jax.lax.broadcasted_iota(
        jnp.int32, s.shape, 1).astype(jnp.float32)
    s = jnp.where(colf < _KB, s, _NEG)

    idx_cols = []
    m0 = None
    denom = None
    s_cur = s
    for j in range(_TOPK):
        m = jnp.max(s_cur, axis=-1, keepdims=True)
        idxf = jnp.min(jnp.where(s_cur == m, colf, 3e9),
                       axis=-1, keepdims=True)
        if j == 0:
            m0 = m
            denom = jnp.ones_like(m)
        else:
            denom = denom + jnp.exp(m - m0)
        idx_cols.append(idxf)
        s_cur = jnp.where(colf == idxf, _NEG, s_cur)

    selected = (s_cur == _NEG) & (colf < _KB)
    walpha = jnp.where(selected, jnp.exp(s - m0) / denom, 0.0)

    eg = jax.lax.dot_general(walpha, vs_ref[...], (((1,), (0,)), ((), ())),
                             preferred_element_type=jnp.float32)
    dv = e_ref.shape[1]
    e_ref[...] = eg[:, :dv]
    g = eg[:, dv:]
    idx_ref[...] = jnp.concatenate(
        [f.astype(jnp.int32) for f in idx_cols]
        + [jnp.zeros((s.shape[0], 8 - _TOPK), jnp.int32)], axis=1)

    csn = c / jnp.maximum(
        jnp.sqrt(jnp.sum(c * c, axis=-1, keepdims=True)), 1e-12)
    part = jnp.sum(1.0 - jnp.sum(csn * g, axis=-1)) * (1.0 / _B)

    @pl.when(i == 0)
    def _():
        cost_ref[...] = jnp.zeros_like(cost_ref)

    cost_ref[...] += part


@jax.jit
def kernel(u_X, c_S, W, b, keys, values, semantic_embeddings):
    b2 = b.reshape(1, -1)
    dk = keys.shape[1]
    dv = values.shape[1]
    dsem = semantic_embeddings.shape[1]

    kn, vs = pl.pallas_call(
        _prep_kern,
        out_shape=[
            jax.ShapeDtypeStruct((_KB_PAD, dk), jnp.float32),
            jax.ShapeDtypeStruct((_KB_PAD, dv + dsem), jnp.float32),
        ],
    )(keys, semantic_embeddings, values)

    e_out, idx_out, cost_out = pl.pallas_call(
        _main_kern,
        grid=(_NBLK,),
        in_specs=[
            pl.BlockSpec((_BLK, u_X.shape[1]), lambda i: (i, 0)),
            pl.BlockSpec((_BLK, c_S.shape[1]), lambda i: (i, 0)),
            pl.BlockSpec(W.shape, lambda i: (0, 0)),
            pl.BlockSpec((1, W.shape[0]), lambda i: (0, 0)),
            pl.BlockSpec((_KB_PAD, dk), lambda i: (0, 0)),
            pl.BlockSpec((_KB_PAD, dv + dsem), lambda i: (0, 0)),
        ],
        out_specs=[
            pl.BlockSpec((_BLK, dv), lambda i: (i, 0)),
            pl.BlockSpec((_BLK, 8), lambda i: (i, 0)),
            pl.BlockSpec((1, 1), lambda i: (0, 0)),
        ],
        out_shape=[
            jax.ShapeDtypeStruct((_B, dv), jnp.float32),
            jax.ShapeDtypeStruct((_B, 8), jnp.int32),
            jax.ShapeDtypeStruct((1, 1), jnp.float32),
        ],
    )(u_X, c_S, W, b2, kn, vs)

    return (e_out, idx_out[:, :_TOPK], cost_out[0, 0])


# speedup vs baseline: 1.5263x; 1.0058x over previous
import jax
import jax.numpy as jnp
from jax.experimental import pallas as pl

_B = 4096
_KB = 1000
_KB_PAD = 1024
_TOPK = 5
_TEMP_INV = 1.0 / 0.07
_BLK = 512
_NBLK = _B // _BLK
_NEG = -1e30


def _prep_kern(keys_ref, sem_ref, val_ref, kn_ref, vs_ref):
    k = keys_ref[...]
    kn = k / jnp.maximum(
        jnp.sqrt(jnp.sum(k * k, axis=-1, keepdims=True)), 1e-12)
    kn_ref[...] = jnp.zeros_like(kn_ref)
    kn_ref[:_KB, :] = kn
    s = sem_ref[...]
    semn = s / jnp.maximum(
        jnp.sqrt(jnp.sum(s * s, axis=-1, keepdims=True)), 1e-12)
    vs_ref[...] = jnp.zeros_like(vs_ref)
    vs_ref[:_KB, :512] = val_ref[...]
    vs_ref[:_KB, 512:] = semn


def _main_kern(u_ref, c_ref, w_ref, b_ref, kn_ref, vs_ref,
               e_ref, idx_ref, cost_ref):
    i = pl.program_id(0)
    w = w_ref[...]
    c = c_ref[...]
    nt = (((1,), (1,)), ((), ()))
    q = (jax.lax.dot_general(u_ref[...], w[:, :512], nt,
                             preferred_element_type=jnp.float32)
         + jax.lax.dot_general(c, w[:, 512:], nt,
                               preferred_element_type=jnp.float32)
         + b_ref[...])
    q = jnp.maximum(q, 0.0)
    qn = q * (1.0 / jnp.maximum(
        jnp.sqrt(jnp.sum(q * q, axis=-1, keepdims=True)), 1e-12))

    s = jax.lax.dot_general(qn, kn_ref[...], nt,
                            preferred_element_type=jnp.float32) * _TEMP_INV
    colf = jax.lax.broadcasted_iota(
        jnp.int32, s.shape, 1).astype(jnp.float32)
    s = jnp.where(colf < _KB, s, _NEG)

    idx_cols = []
    m0 = None
    denom = None
    s_cur = s
    for j in range(_TOPK):
        m = jnp.max(s_cur, axis=-1, keepdims=True)
        idxf = jnp.min(jnp.where(s_cur == m, colf, 3e9),
                       axis=-1, keepdims=True)
        if j == 0:
            m0 = m
            denom = jnp.ones_like(m)
        else:
            denom = denom + jnp.exp(m - m0)
        idx_cols.append(idxf)
        s_cur = jnp.where(colf == idxf, _NEG, s_cur)

    selected = (s_cur == _NEG) & (colf < _KB)
    walpha = jnp.where(selected, jnp.exp(s - m0) * (1.0 / denom), 0.0)

    eg = jax.lax.dot_general(walpha, vs_ref[...], (((1,), (0,)), ((), ())),
                             preferred_element_type=jnp.float32)
    dv = e_ref.shape[1]
    e_ref[...] = eg[:, :dv]
    g = eg[:, dv:]
    idx_ref[...] = jnp.concatenate(
        [f.astype(jnp.int32) for f in idx_cols]
        + [jnp.zeros((s.shape[0], 8 - _TOPK), jnp.int32)], axis=1)

    csn = c * (1.0 / jnp.maximum(
        jnp.sqrt(jnp.sum(c * c, axis=-1, keepdims=True)), 1e-12))
    part = jnp.sum(1.0 - jnp.sum(csn * g, axis=-1)) * (1.0 / _B)

    @pl.when(i == 0)
    def _():
        cost_ref[...] = jnp.zeros_like(cost_ref)

    cost_ref[...] += part


@jax.jit
def kernel(u_X, c_S, W, b, keys, values, semantic_embeddings):
    b2 = b.reshape(1, -1)
    dk = keys.shape[1]
    dv = values.shape[1]
    dsem = semantic_embeddings.shape[1]

    kn, vs = pl.pallas_call(
        _prep_kern,
        out_shape=[
            jax.ShapeDtypeStruct((_KB_PAD, dk), jnp.float32),
            jax.ShapeDtypeStruct((_KB_PAD, dv + dsem), jnp.float32),
        ],
    )(keys, semantic_embeddings, values)

    e_out, idx_out, cost_out = pl.pallas_call(
        _main_kern,
        grid=(_NBLK,),
        in_specs=[
            pl.BlockSpec((_BLK, u_X.shape[1]), lambda i: (i, 0)),
            pl.BlockSpec((_BLK, c_S.shape[1]), lambda i: (i, 0)),
            pl.BlockSpec(W.shape, lambda i: (0, 0)),
            pl.BlockSpec((1, W.shape[0]), lambda i: (0, 0)),
            pl.BlockSpec((_KB_PAD, dk), lambda i: (0, 0)),
            pl.BlockSpec((_KB_PAD, dv + dsem), lambda i: (0, 0)),
        ],
        out_specs=[
            pl.BlockSpec((_BLK, dv), lambda i: (i, 0)),
            pl.BlockSpec((_BLK, 8), lambda i: (i, 0)),
            pl.BlockSpec((1, 1), lambda i: (0, 0)),
        ],
        out_shape=[
            jax.ShapeDtypeStruct((_B, dv), jnp.float32),
            jax.ShapeDtypeStruct((_B, 8), jnp.int32),
            jax.ShapeDtypeStruct((1, 1), jnp.float32),
        ],
    )(u_X, c_S, W, b2, kn, vs)

    return (e_out, idx_out[:, :_TOPK], cost_out[0, 0])


# speedup vs baseline: 1.6749x; 1.0974x over previous
import jax
import jax.numpy as jnp
from jax.experimental import pallas as pl
from jax.experimental.pallas import tpu as pltpu

_B = 4096
_KB = 1000
_KB_PAD = 1024
_TOPK = 5
_TEMP_INV = 1.0 / 0.07
_BLK = 512
_NBLK = _B // _BLK
_NEG = -1e30


def _main_kern(u_ref, c_ref, w_ref, b_ref, keys_ref, sem_ref, val_ref,
               e_ref, idx_ref, cost_ref, kn_ref, vs_ref):
    i = pl.program_id(0)

    @pl.when(i == 0)
    def _():
        k = keys_ref[...]
        kn = k * (1.0 / jnp.maximum(
            jnp.sqrt(jnp.sum(k * k, axis=-1, keepdims=True)), 1e-12))
        kn_ref[...] = jnp.zeros_like(kn_ref)
        kn_ref[:_KB, :] = kn
        sm = sem_ref[...]
        semn = sm * (1.0 / jnp.maximum(
            jnp.sqrt(jnp.sum(sm * sm, axis=-1, keepdims=True)), 1e-12))
        vs_ref[...] = jnp.zeros_like(vs_ref)
        vs_ref[:_KB, :512] = val_ref[...]
        vs_ref[:_KB, 512:] = semn
        cost_ref[...] = jnp.zeros_like(cost_ref)

    w = w_ref[...]
    c = c_ref[...]
    nt = (((1,), (1,)), ((), ()))
    q = (jax.lax.dot_general(u_ref[...], w[:, :512], nt,
                             preferred_element_type=jnp.float32)
         + jax.lax.dot_general(c, w[:, 512:], nt,
                               preferred_element_type=jnp.float32)
         + b_ref[...])
    q = jnp.maximum(q, 0.0)
    qn = q * (1.0 / jnp.maximum(
        jnp.sqrt(jnp.sum(q * q, axis=-1, keepdims=True)), 1e-12))

    s = jax.lax.dot_general(qn, kn_ref[...], nt,
                            preferred_element_type=jnp.float32) * _TEMP_INV
    colf = jax.lax.broadcasted_iota(
        jnp.int32, s.shape, 1).astype(jnp.float32)
    s = jnp.where(colf < _KB, s, _NEG)

    idx_cols = []
    m0 = None
    denom = None
    s_cur = s
    for j in range(_TOPK):
        m = jnp.max(s_cur, axis=-1, keepdims=True)
        idxf = jnp.min(jnp.where(s_cur == m, colf, 3e9),
                       axis=-1, keepdims=True)
        if j == 0:
            m0 = m
            denom = jnp.ones_like(m)
        else:
            denom = denom + jnp.exp(m - m0)
        idx_cols.append(idxf)
        s_cur = jnp.where(colf == idxf, _NEG, s_cur)

    selected = (s_cur == _NEG) & (colf < _KB)
    walpha = jnp.where(selected, jnp.exp(s - m0) * (1.0 / denom), 0.0)

    eg = jax.lax.dot_general(walpha, vs_ref[...], (((1,), (0,)), ((), ())),
                             preferred_element_type=jnp.float32)
    dv = e_ref.shape[1]
    e_ref[...] = eg[:, :dv]
    g = eg[:, dv:]
    idx_ref[...] = jnp.concatenate(
        [f.astype(jnp.int32) for f in idx_cols]
        + [jnp.zeros((s.shape[0], 8 - _TOPK), jnp.int32)], axis=1)

    csn = c * (1.0 / jnp.maximum(
        jnp.sqrt(jnp.sum(c * c, axis=-1, keepdims=True)), 1e-12))
    part = jnp.sum(1.0 - jnp.sum(csn * g, axis=-1)) * (1.0 / _B)
    cost_ref[...] += part


@jax.jit
def kernel(u_X, c_S, W, b, keys, values, semantic_embeddings):
    b2 = b.reshape(1, -1)
    dk = keys.shape[1]
    dv = values.shape[1]
    dsem = semantic_embeddings.shape[1]

    e_out, idx_out, cost_out = pl.pallas_call(
        _main_kern,
        grid=(_NBLK,),
        in_specs=[
            pl.BlockSpec((_BLK, u_X.shape[1]), lambda i: (i, 0)),
            pl.BlockSpec((_BLK, c_S.shape[1]), lambda i: (i, 0)),
            pl.BlockSpec(W.shape, lambda i: (0, 0)),
            pl.BlockSpec((1, W.shape[0]), lambda i: (0, 0)),
            pl.BlockSpec(keys.shape, lambda i: (0, 0)),
            pl.BlockSpec(semantic_embeddings.shape, lambda i: (0, 0)),
            pl.BlockSpec(values.shape, lambda i: (0, 0)),
        ],
        out_specs=[
            pl.BlockSpec((_BLK, dv), lambda i: (i, 0)),
            pl.BlockSpec((_BLK, 8), lambda i: (i, 0)),
            pl.BlockSpec((1, 1), lambda i: (0, 0)),
        ],
        out_shape=[
            jax.ShapeDtypeStruct((_B, dv), jnp.float32),
            jax.ShapeDtypeStruct((_B, 8), jnp.int32),
            jax.ShapeDtypeStruct((1, 1), jnp.float32),
        ],
        scratch_shapes=[
            pltpu.VMEM((_KB_PAD, dk), jnp.float32),
            pltpu.VMEM((_KB_PAD, dv + dsem), jnp.float32),
        ],
    )(u_X, c_S, W, b2, keys, semantic_embeddings, values)

    return (e_out, idx_out[:, :_TOPK], cost_out[0, 0])


# speedup vs baseline: 1.7387x; 1.0381x over previous
import jax
import jax.numpy as jnp
from jax.experimental import pallas as pl
from jax.experimental.pallas import tpu as pltpu

_B = 4096
_KB = 1000
_KB_PAD = 1024
_TOPK = 5
_TEMP_INV = 1.0 / 0.07
_BLK = 1024
_NBLK = _B // _BLK
_NEG = -1e30


def _main_kern(u_ref, c_ref, w_ref, b_ref, keys_ref, sem_ref, val_ref,
               e_ref, idx_ref, cost_ref, kn_ref, vs_ref):
    i = pl.program_id(0)

    @pl.when(i == 0)
    def _():
        k = keys_ref[...]
        kn = k * (1.0 / jnp.maximum(
            jnp.sqrt(jnp.sum(k * k, axis=-1, keepdims=True)), 1e-12))
        kn_ref[...] = jnp.zeros_like(kn_ref)
        kn_ref[:_KB, :] = kn
        sm = sem_ref[...]
        semn = sm * (1.0 / jnp.maximum(
            jnp.sqrt(jnp.sum(sm * sm, axis=-1, keepdims=True)), 1e-12))
        vs_ref[...] = jnp.zeros_like(vs_ref)
        vs_ref[:_KB, :512] = val_ref[...]
        vs_ref[:_KB, 512:] = semn
        cost_ref[...] = jnp.zeros_like(cost_ref)

    w = w_ref[...]
    c = c_ref[...]
    nt = (((1,), (1,)), ((), ()))
    q = (jax.lax.dot_general(u_ref[...], w[:, :512], nt,
                             preferred_element_type=jnp.float32)
         + jax.lax.dot_general(c, w[:, 512:], nt,
                               preferred_element_type=jnp.float32)
         + b_ref[...])
    q = jnp.maximum(q, 0.0)
    qn = q * (1.0 / jnp.maximum(
        jnp.sqrt(jnp.sum(q * q, axis=-1, keepdims=True)), 1e-12))

    s = jax.lax.dot_general(qn, kn_ref[...], nt,
                            preferred_element_type=jnp.float32) * _TEMP_INV
    colf = jax.lax.broadcasted_iota(
        jnp.int32, s.shape, 1).astype(jnp.float32)
    s = jnp.where(colf < _KB, s, _NEG)

    idx_cols = []
    m0 = None
    denom = None
    s_cur = s
    for j in range(_TOPK):
        m = jnp.max(s_cur, axis=-1, keepdims=True)
        idxf = jnp.min(jnp.where(s_cur == m, colf, 3e9),
                       axis=-1, keepdims=True)
        if j == 0:
            m0 = m
            denom = jnp.ones_like(m)
        else:
            denom = denom + jnp.exp(m - m0)
        idx_cols.append(idxf)
        s_cur = jnp.where(colf == idxf, _NEG, s_cur)

    selected = (s_cur == _NEG) & (colf < _KB)
    walpha = jnp.where(selected, jnp.exp(s - m0) * (1.0 / denom), 0.0)

    eg = jax.lax.dot_general(walpha, vs_ref[...], (((1,), (0,)), ((), ())),
                             preferred_element_type=jnp.float32)
    dv = e_ref.shape[1]
    e_ref[...] = eg[:, :dv]
    g = eg[:, dv:]
    idx_ref[...] = jnp.concatenate(
        [f.astype(jnp.int32) for f in idx_cols]
        + [jnp.zeros((s.shape[0], 8 - _TOPK), jnp.int32)], axis=1)

    csn = c * (1.0 / jnp.maximum(
        jnp.sqrt(jnp.sum(c * c, axis=-1, keepdims=True)), 1e-12))
    part = jnp.sum(1.0 - jnp.sum(csn * g, axis=-1)) * (1.0 / _B)
    cost_ref[...] += part


@jax.jit
def kernel(u_X, c_S, W, b, keys, values, semantic_embeddings):
    b2 = b.reshape(1, -1)
    dk = keys.shape[1]
    dv = values.shape[1]
    dsem = semantic_embeddings.shape[1]

    e_out, idx_out, cost_out = pl.pallas_call(
        _main_kern,
        grid=(_NBLK,),
        in_specs=[
            pl.BlockSpec((_BLK, u_X.shape[1]), lambda i: (i, 0)),
            pl.BlockSpec((_BLK, c_S.shape[1]), lambda i: (i, 0)),
            pl.BlockSpec(W.shape, lambda i: (0, 0)),
            pl.BlockSpec((1, W.shape[0]), lambda i: (0, 0)),
            pl.BlockSpec(keys.shape, lambda i: (0, 0)),
            pl.BlockSpec(semantic_embeddings.shape, lambda i: (0, 0)),
            pl.BlockSpec(values.shape, lambda i: (0, 0)),
        ],
        out_specs=[
            pl.BlockSpec((_BLK, dv), lambda i: (i, 0)),
            pl.BlockSpec((_BLK, 8), lambda i: (i, 0)),
            pl.BlockSpec((1, 1), lambda i: (0, 0)),
        ],
        out_shape=[
            jax.ShapeDtypeStruct((_B, dv), jnp.float32),
            jax.ShapeDtypeStruct((_B, 8), jnp.int32),
            jax.ShapeDtypeStruct((1, 1), jnp.float32),
        ],
        scratch_shapes=[
            pltpu.VMEM((_KB_PAD, dk), jnp.float32),
            pltpu.VMEM((_KB_PAD, dv + dsem), jnp.float32),
        ],
    )(u_X, c_S, W, b2, keys, semantic_embeddings, values)

    return (e_out, idx_out[:, :_TOPK], cost_out[0, 0])
